# Initial kernel scaffold; baseline (speedup 1.0000x reference)
import jax
import jax.numpy as jnp
from jax.experimental import pallas as pl


def kernel(x, edge_index, W1, b1, W2, b2, pairwise_weights):
    raise NotImplementedError("write your pallas kernel here")


# speedup vs baseline: 13.2298x; 13.2298x over previous
import functools

import jax
import jax.numpy as jnp
from jax import lax
from jax.experimental import pallas as pl
from jax.experimental.pallas import tpu as pltpu
from jax.experimental.pallas import tpu_sc as plsc

N = 100000
D = 256
C = 6
CP = 8
BP_ITERS = 5

NUM_WORKERS = 32
LANE = 128
RB = 8
R_PER_W = 392
RTOT = NUM_WORKERS * R_PER_W
E_PAD = RTOT * LANE
NP = 100352
ROWS_PER_SUB = NP // 16
BLK = 2000


def _mlp_body(x_ref, w1_ref, b1_ref, w2_ref, b2_ref, pt_ref, logu_ref, logpb_ref):
    h = jnp.maximum(
        jnp.dot(x_ref[...], w1_ref[...], preferred_element_type=jnp.float32)
        + b1_ref[...], 0.0)
    u = (jnp.dot(h, w2_ref[...], preferred_element_type=jnp.float32)
         + b2_ref[...])
    m = jnp.max(u, axis=1, keepdims=True)
    eu = jnp.exp(u - m)
    s = jnp.sum(eu, axis=1, keepdims=True)
    logu_ref[...] = u - m - jnp.log(s)
    bel = eu / s
    belb = bel.astype(jnp.bfloat16).astype(jnp.float32)
    mm = jnp.zeros_like(bel)
    for c in range(C):
        mm = mm + belb[:, c:c + 1] * pt_ref[c:c + 1, :]
    lane = lax.broadcasted_iota(jnp.int32, mm.shape, 1)
    logpb_ref[...] = jnp.where(lane < C, jnp.log(mm), 0.0)


def _combine_body(logu_ref, p0_ref, p1_ref, pt_ref, bel_ref, logpb_ref):
    lb = logu_ref[...] + p0_ref[...] + p1_ref[...]
    m = jnp.max(lb, axis=1, keepdims=True)
    e = jnp.exp(lb - m)
    s = jnp.sum(e, axis=1, keepdims=True)
    bel = e / (s + 1e-10)
    bel_ref[...] = bel
    belb = bel.astype(jnp.bfloat16).astype(jnp.float32)
    mm = jnp.zeros_like(bel)
    for c in range(C):
        mm = mm + belb[:, c:c + 1] * pt_ref[c:c + 1, :]
    lane = lax.broadcasted_iota(jnp.int32, mm.shape, 1)
    logpb_ref[...] = jnp.where(lane < C, jnp.log(mm), 0.0)


def _edge_body(src_hbm, dst_hbm, logpb_hbm, zeros_hbm, out_hbm,
               src_v, dst_v, rows_v, acc_sh, gsem):
    cid = lax.axis_index("c")
    sid = lax.axis_index("s")
    w = sid * 2 + cid
    pltpu.sync_copy(zeros_hbm.at[pl.ds(sid * ROWS_PER_SUB, ROWS_PER_SUB)],
                    acc_sh.at[pl.ds(sid * ROWS_PER_SUB, ROWS_PER_SUB)])
    plsc.subcore_barrier()

    base_row = w * R_PER_W

    @pl.loop(0, R_PER_W // RB)
    def _step(i):
        r0 = base_row + i * RB
        pltpu.sync_copy(src_hbm.at[pl.ds(r0, RB)], src_v)
        pltpu.sync_copy(dst_hbm.at[pl.ds(r0, RB)], dst_v)
        cps = [pltpu.async_copy(logpb_hbm.at[src_v.at[j]],
                                rows_v.at[pl.ds(j * LANE, LANE)], gsem)
               for j in range(RB)]
        for cp in cps:
            cp.wait()
        for j in range(RB):
            pltpu.sync_copy(rows_v.at[pl.ds(j * LANE, LANE)],
                            acc_sh.at[dst_v.at[j]], add=True)

    plsc.subcore_barrier()
    pltpu.sync_copy(acc_sh.at[pl.ds(sid * ROWS_PER_SUB, ROWS_PER_SUB)],
                    out_hbm.at[cid, pl.ds(sid * ROWS_PER_SUB, ROWS_PER_SUB)])


def _build_calls():
    common = dict(
        w1=pl.BlockSpec((D, 128), lambda i: (0, 0)),
        b1=pl.BlockSpec((1, 128), lambda i: (0, 0)),
        w2=pl.BlockSpec((128, CP), lambda i: (0, 0)),
        b2=pl.BlockSpec((1, CP), lambda i: (0, 0)),
        pt=pl.BlockSpec((CP, CP), lambda i: (0, 0)),
        row=pl.BlockSpec((BLK, CP), lambda i: (i, 0)),
    )
    mlp = pl.pallas_call(
        _mlp_body,
        grid=(N // BLK,),
        in_specs=[pl.BlockSpec((BLK, D), lambda i: (i, 0)),
                  common["w1"], common["b1"], common["w2"], common["b2"],
                  common["pt"]],
        out_specs=[common["row"], common["row"]],
        out_shape=[jax.ShapeDtypeStruct((N, CP), jnp.float32),
                   jax.ShapeDtypeStruct((N, CP), jnp.float32)],
    )
    combine = pl.pallas_call(
        _combine_body,
        grid=(N // BLK,),
        in_specs=[common["row"], common["row"], common["row"], common["pt"]],
        out_specs=[common["row"], common["row"]],
        out_shape=[jax.ShapeDtypeStruct((N, CP), jnp.float32),
                   jax.ShapeDtypeStruct((N, CP), jnp.float32)],
    )
    return mlp, combine


_MLP, _COMBINE = _build_calls()


@functools.cache
def _edge_call():
    mesh = plsc.VectorSubcoreMesh(core_axis_name="c", subcore_axis_name="s")
    return pl.kernel(
        _edge_body,
        out_type=jax.ShapeDtypeStruct((2, NP, CP), jnp.float32),
        mesh=mesh,
        compiler_params=pltpu.CompilerParams(use_tc_tiling_on_sc=False),
        scratch_types=[
            pltpu.VMEM((RB, LANE), jnp.int32),
            pltpu.VMEM((RB, LANE), jnp.int32),
            pltpu.VMEM((RB * LANE, CP), jnp.float32),
            pltpu.VMEM_SHARED((NP, CP), jnp.float32),
            pltpu.SemaphoreType.DMA,
        ],
    )


def kernel(x, edge_index, W1, b1, W2, b2, pairwise_weights):
    src = edge_index[0]
    dst = edge_index[1]
    e = src.shape[0]
    src_r = jnp.concatenate(
        [src, jnp.zeros((E_PAD - e,), jnp.int32)]).reshape(RTOT, LANE)
    dst_r = jnp.concatenate(
        [dst, jnp.full((E_PAD - e,), N, jnp.int32)]).reshape(RTOT, LANE)

    b1r = b1.reshape(1, 128)
    w2p = jnp.pad(W2, ((0, 0), (0, CP - C)))
    b2p = jnp.concatenate([b2, jnp.full((CP - C,), -1e30, jnp.float32)]
                          ).reshape(1, CP)
    pt8 = jnp.zeros((CP, CP), jnp.float32).at[:C, :C].set(
        pairwise_weights.T.astype(jnp.bfloat16).astype(jnp.float32))
    zeros_np = jnp.zeros((NP, CP), jnp.float32)

    logu, logpb = _MLP(x, W1, b1r, w2p, b2p, pt8)
    edge = _edge_call()
    bel = None
    for _ in range(BP_ITERS):
        parts = edge(src_r, dst_r, logpb, zeros_np)
        bel, logpb = _COMBINE(logu, parts[0, :N], parts[1, :N], pt8)
    return bel[:, :C]


# speedup vs baseline: 16.0655x; 1.2143x over previous
import functools

import jax
import jax.numpy as jnp
from jax import lax
from jax.experimental import pallas as pl
from jax.experimental.pallas import tpu as pltpu
from jax.experimental.pallas import tpu_sc as plsc

N = 100000
D = 256
C = 6
CP = 8
BP_ITERS = 5

NUM_WORKERS = 32
LANE = 128
RB = 14
NSTEP = 28
R_PER_W = NSTEP * RB
RTOT = NUM_WORKERS * R_PER_W
E_PAD = RTOT * LANE
NP = 100352
ROWS_PER_SUB = NP // 16
BLK = 2000


def _mlp_body(x_ref, w1_ref, b1_ref, w2_ref, b2_ref, pt_ref, logu_ref, logpb_ref):
    h = jnp.maximum(
        jnp.dot(x_ref[...], w1_ref[...], preferred_element_type=jnp.float32)
        + b1_ref[...], 0.0)
    u = (jnp.dot(h, w2_ref[...], preferred_element_type=jnp.float32)
         + b2_ref[...])
    m = jnp.max(u, axis=1, keepdims=True)
    eu = jnp.exp(u - m)
    s = jnp.sum(eu, axis=1, keepdims=True)
    logu_ref[...] = u - m - jnp.log(s)
    bel = eu / s
    belb = bel.astype(jnp.bfloat16).astype(jnp.float32)
    mm = jnp.zeros_like(bel)
    for c in range(C):
        mm = mm + belb[:, c:c + 1] * pt_ref[c:c + 1, :]
    lane = lax.broadcasted_iota(jnp.int32, mm.shape, 1)
    logpb_ref[...] = jnp.where(lane < C, jnp.log(mm), 0.0)


def _combine_body(logu_ref, p0_ref, p1_ref, pt_ref, bel_ref, logpb_ref):
    lb = logu_ref[...] + p0_ref[...] + p1_ref[...]
    m = jnp.max(lb, axis=1, keepdims=True)
    e = jnp.exp(lb - m)
    s = jnp.sum(e, axis=1, keepdims=True)
    bel = e / (s + 1e-10)
    bel_ref[...] = bel
    belb = bel.astype(jnp.bfloat16).astype(jnp.float32)
    mm = jnp.zeros_like(bel)
    for c in range(C):
        mm = mm + belb[:, c:c + 1] * pt_ref[c:c + 1, :]
    lane = lax.broadcasted_iota(jnp.int32, mm.shape, 1)
    logpb_ref[...] = jnp.where(lane < C, jnp.log(mm), 0.0)


def _edge_body(src_hbm, dst_hbm, logpb_hbm, zeros_hbm, out_hbm,
               src_q0, src_q1, src_q2, src_q3,
               dst_q0, dst_q1, dst_q2, dst_q3,
               rows_r0, rows_r1, acc_sh,
               isem0, isem1, isem2, isem3, gsem0, gsem1, ssem0, ssem1):
    cid = lax.axis_index("c")
    sid = lax.axis_index("s")
    w = sid * 2 + cid
    src_q = (src_q0, src_q1, src_q2, src_q3)
    dst_q = (dst_q0, dst_q1, dst_q2, dst_q3)
    rows = (rows_r0, rows_r1)
    isem = (isem0, isem1, isem2, isem3)
    gsem = (gsem0, gsem1)
    ssem = (ssem0, ssem1)

    pltpu.sync_copy(zeros_hbm.at[pl.ds(sid * ROWS_PER_SUB, ROWS_PER_SUB)],
                    acc_sh.at[pl.ds(sid * ROWS_PER_SUB, ROWS_PER_SUB)])
    plsc.subcore_barrier()

    base_row = w * R_PER_W

    def fire_idx(b, q):
        r0 = base_row + b * RB
        pltpu.async_copy(src_hbm.at[pl.ds(r0, RB)], src_q[q], isem[q])
        pltpu.async_copy(dst_hbm.at[pl.ds(r0, RB)], dst_q[q], isem[q])

    def wait_idx(q):
        pltpu.make_async_copy(src_hbm.at[pl.ds(base_row, RB)],
                              src_q[q], isem[q]).wait()
        pltpu.make_async_copy(dst_hbm.at[pl.ds(base_row, RB)],
                              dst_q[q], isem[q]).wait()

    def fire_gathers(r, q):
        for j in range(RB):
            pltpu.async_copy(logpb_hbm.at[src_q[q].at[j]],
                             rows[r].at[pl.ds(j * LANE, LANE)], gsem[r])

    def drain_gathers(r, q):
        for j in range(RB):
            pltpu.make_async_copy(logpb_hbm.at[src_q[q].at[j]],
                                  rows[r].at[pl.ds(j * LANE, LANE)],
                                  gsem[r]).wait()

    def fire_scatters(r, q):
        for j in range(RB):
            pltpu.async_copy(rows[r].at[pl.ds(j * LANE, LANE)],
                             acc_sh.at[dst_q[q].at[j]], ssem[r], add=True)

    def drain_scatters(r, q):
        for j in range(RB):
            pltpu.make_async_copy(rows[r].at[pl.ds(j * LANE, LANE)],
                                  acc_sh.at[dst_q[q].at[j]], ssem[r]).wait()

    fire_idx(0, 0)
    wait_idx(0)
    fire_gathers(0, 0)
    fire_idx(1, 1)

    @pl.loop(0, NSTEP // 4)
    def _step(i):
        a = 4 * i
        for k in range(4):
            b = a + k
            r, q = k % 2, k % 4
            drain_gathers(r, q)

            @pl.when(b > 0)
            def _():
                drain_scatters(1 - r, (q - 1) % 4)
            fire_scatters(r, q)

            @pl.when(b + 1 < NSTEP)
            def _():
                wait_idx((q + 1) % 4)

            @pl.when(b + 2 < NSTEP)
            def _():
                fire_idx(b + 2, (q + 2) % 4)

            @pl.when(b + 1 < NSTEP)
            def _():
                fire_gathers(1 - r, (q + 1) % 4)

    drain_scatters((NSTEP - 1) % 2, (NSTEP - 1) % 4)
    plsc.subcore_barrier()
    pltpu.sync_copy(acc_sh.at[pl.ds(sid * ROWS_PER_SUB, ROWS_PER_SUB)],
                    out_hbm.at[cid, pl.ds(sid * ROWS_PER_SUB, ROWS_PER_SUB)])


def _build_calls():
    common = dict(
        w1=pl.BlockSpec((D, 128), lambda i: (0, 0)),
        b1=pl.BlockSpec((1, 128), lambda i: (0, 0)),
        w2=pl.BlockSpec((128, CP), lambda i: (0, 0)),
        b2=pl.BlockSpec((1, CP), lambda i: (0, 0)),
        pt=pl.BlockSpec((CP, CP), lambda i: (0, 0)),
        row=pl.BlockSpec((BLK, CP), lambda i: (i, 0)),
    )
    mlp = pl.pallas_call(
        _mlp_body,
        grid=(N // BLK,),
        in_specs=[pl.BlockSpec((BLK, D), lambda i: (i, 0)),
                  common["w1"], common["b1"], common["w2"], common["b2"],
                  common["pt"]],
        out_specs=[common["row"], common["row"]],
        out_shape=[jax.ShapeDtypeStruct((N, CP), jnp.float32),
                   jax.ShapeDtypeStruct((N, CP), jnp.float32)],
    )
    combine = pl.pallas_call(
        _combine_body,
        grid=(N // BLK,),
        in_specs=[common["row"], common["row"], common["row"], common["pt"]],
        out_specs=[common["row"], common["row"]],
        out_shape=[jax.ShapeDtypeStruct((N, CP), jnp.float32),
                   jax.ShapeDtypeStruct((N, CP), jnp.float32)],
    )
    return mlp, combine


_MLP, _COMBINE = _build_calls()


@functools.cache
def _edge_call():
    mesh = plsc.VectorSubcoreMesh(core_axis_name="c", subcore_axis_name="s")
    return pl.kernel(
        _edge_body,
        out_type=jax.ShapeDtypeStruct((2, NP, CP), jnp.float32),
        mesh=mesh,
        compiler_params=pltpu.CompilerParams(use_tc_tiling_on_sc=False),
        scratch_types=(
            [pltpu.VMEM((RB, LANE), jnp.int32)] * 8
            + [pltpu.VMEM((RB * LANE, CP), jnp.float32)] * 2
            + [pltpu.VMEM_SHARED((NP, CP), jnp.float32)]
            + [pltpu.SemaphoreType.DMA] * 8
        ),
    )


def kernel(x, edge_index, W1, b1, W2, b2, pairwise_weights):
    src = edge_index[0]
    dst = edge_index[1]
    e = src.shape[0]
    src_r = jnp.concatenate(
        [src, jnp.zeros((E_PAD - e,), jnp.int32)]).reshape(RTOT, LANE)
    dst_r = jnp.concatenate(
        [dst, jnp.full((E_PAD - e,), N, jnp.int32)]).reshape(RTOT, LANE)

    b1r = b1.reshape(1, 128)
    w2p = jnp.pad(W2, ((0, 0), (0, CP - C)))
    b2p = jnp.concatenate([b2, jnp.full((CP - C,), -1e30, jnp.float32)]
                          ).reshape(1, CP)
    pt8 = jnp.zeros((CP, CP), jnp.float32).at[:C, :C].set(
        pairwise_weights.T.astype(jnp.bfloat16).astype(jnp.float32))
    zeros_np = jnp.zeros((NP, CP), jnp.float32)

    logu, logpb = _MLP(x, W1, b1r, w2p, b2p, pt8)
    edge = _edge_call()
    bel = None
    for _ in range(BP_ITERS):
        parts = edge(src_r, dst_r, logpb, zeros_np)
        bel, logpb = _COMBINE(logu, parts[0, :N], parts[1, :N], pt8)
    return bel[:, :C]


# speedup vs baseline: 35.3051x; 2.1976x over previous
import functools

import jax
import jax.numpy as jnp
from jax import lax
from jax.experimental import pallas as pl
from jax.experimental.pallas import tpu as pltpu
from jax.experimental.pallas import tpu_sc as plsc

N = 100000
D = 256
C = 6
CP = 8
BP_ITERS = 5

NUM_WORKERS = 32
LANE = 128
RB = 14
NSTEP = 28
R_PER_W = NSTEP * RB
RTOT = NUM_WORKERS * R_PER_W
E_PAD = RTOT * LANE
NP = 100352
ROWS_PER_SUB = NP // 16
MBLK = 5000
NPACK = N // 16


def _mlp_body(x_ref, w1_ref, b1_ref, w2_ref, b2_ref, pt_ref, logu_ref, logpb_ref):
    h = jnp.maximum(
        jnp.dot(x_ref[...], w1_ref[...], preferred_element_type=jnp.float32)
        + b1_ref[...], 0.0)
    u = (jnp.dot(h, w2_ref[...], preferred_element_type=jnp.float32)
         + b2_ref[...])
    m = jnp.max(u, axis=1, keepdims=True)
    eu = jnp.exp(u - m)
    s = jnp.sum(eu, axis=1, keepdims=True)
    logu_ref[...] = u - m - jnp.log(s)
    bel = eu / s
    belb = bel.astype(jnp.bfloat16).astype(jnp.float32)
    mm = jnp.zeros_like(bel)
    for c in range(C):
        mm = mm + belb[:, c:c + 1] * pt_ref[c:c + 1, :]
    lane = lax.broadcasted_iota(jnp.int32, mm.shape, 1)
    logpb_ref[...] = jnp.where(lane < C, jnp.log(mm), 0.0)


def _combine_body(logu_ref, parts_ref, smat_ref, bd_ref, bel_ref, logpb_ref):
    lb = logu_ref[...] + parts_ref[0, :NPACK] + parts_ref[1, :NPACK]
    y = jnp.maximum(lb, pltpu.roll(lb, 127, 1))
    y = jnp.maximum(y, pltpu.roll(y, 126, 1))
    y = jnp.maximum(y, pltpu.roll(y, 124, 1))
    lane = lax.broadcasted_iota(jnp.int32, lb.shape, 1)
    leaders = jnp.where(lane % 8 == 0, y, 0.0)
    m = jnp.dot(leaders, smat_ref[...], preferred_element_type=jnp.float32)
    e = jnp.exp(lb - m)
    s = jnp.dot(e, smat_ref[...], preferred_element_type=jnp.float32)
    bel = e / (s + 1e-10)
    bel_ref[...] = bel
    belb = bel.astype(jnp.bfloat16).astype(jnp.float32)
    mm = jnp.dot(belb, bd_ref[...], preferred_element_type=jnp.float32)
    logpb_ref[...] = jnp.where(lane % 8 < C, jnp.log(mm), 0.0)


def _edge_body(src_hbm, dst_hbm, logpb_hbm, zeros_hbm, out_hbm,
               src_q0, src_q1, src_q2, src_q3,
               dst_q0, dst_q1, dst_q2, dst_q3,
               rows_r0, rows_r1, acc_sh,
               isem0, isem1, isem2, isem3, gsem0, gsem1, ssem0, ssem1):
    cid = lax.axis_index("c")
    sid = lax.axis_index("s")
    w = sid * 2 + cid
    src_q = (src_q0, src_q1, src_q2, src_q3)
    dst_q = (dst_q0, dst_q1, dst_q2, dst_q3)
    rows = (rows_r0, rows_r1)
    isem = (isem0, isem1, isem2, isem3)
    gsem = (gsem0, gsem1)
    ssem = (ssem0, ssem1)

    pltpu.sync_copy(zeros_hbm.at[pl.ds(sid * ROWS_PER_SUB, ROWS_PER_SUB)],
                    acc_sh.at[pl.ds(sid * ROWS_PER_SUB, ROWS_PER_SUB)])
    plsc.subcore_barrier()

    base_row = w * R_PER_W

    def fire_idx(b, q):
        r0 = base_row + b * RB
        pltpu.async_copy(src_hbm.at[pl.ds(r0, RB)], src_q[q], isem[q])
        pltpu.async_copy(dst_hbm.at[pl.ds(r0, RB)], dst_q[q], isem[q])

    def wait_idx(q):
        pltpu.make_async_copy(src_hbm.at[pl.ds(base_row, RB)],
                              src_q[q], isem[q]).wait()
        pltpu.make_async_copy(dst_hbm.at[pl.ds(base_row, RB)],
                              dst_q[q], isem[q]).wait()

    def fire_gathers(r, q):
        for j in range(RB):
            pltpu.async_copy(logpb_hbm.at[src_q[q].at[j]],
                             rows[r].at[pl.ds(j * LANE, LANE)], gsem[r])

    def drain_gathers(r, q):
        for j in range(RB):
            pltpu.make_async_copy(logpb_hbm.at[src_q[q].at[j]],
                                  rows[r].at[pl.ds(j * LANE, LANE)],
                                  gsem[r]).wait()

    def fire_scatters(r, q):
        for j in range(RB):
            pltpu.async_copy(rows[r].at[pl.ds(j * LANE, LANE)],
                             acc_sh.at[dst_q[q].at[j]], ssem[r], add=True)

    def drain_scatters(r, q):
        for j in range(RB):
            pltpu.make_async_copy(rows[r].at[pl.ds(j * LANE, LANE)],
                                  acc_sh.at[dst_q[q].at[j]], ssem[r]).wait()

    fire_idx(0, 0)
    wait_idx(0)
    fire_gathers(0, 0)
    fire_idx(1, 1)

    @pl.loop(0, NSTEP // 4)
    def _step(i):
        a = 4 * i
        for k in range(4):
            b = a + k
            r, q = k % 2, k % 4
            drain_gathers(r, q)

            @pl.when(b > 0)
            def _():
                drain_scatters(1 - r, (q - 1) % 4)
            fire_scatters(r, q)

            @pl.when(b + 1 < NSTEP)
            def _():
                wait_idx((q + 1) % 4)

            @pl.when(b + 2 < NSTEP)
            def _():
                fire_idx(b + 2, (q + 2) % 4)

            @pl.when(b + 1 < NSTEP)
            def _():
                fire_gathers(1 - r, (q + 1) % 4)

    drain_scatters((NSTEP - 1) % 2, (NSTEP - 1) % 4)
    plsc.subcore_barrier()
    pltpu.sync_copy(acc_sh.at[pl.ds(sid * ROWS_PER_SUB, ROWS_PER_SUB)],
                    out_hbm.at[cid, pl.ds(sid * ROWS_PER_SUB, ROWS_PER_SUB)])


def _build_calls():
    small = dict(
        w1=pl.BlockSpec((D, 128), lambda i: (0, 0)),
        b1=pl.BlockSpec((1, 128), lambda i: (0, 0)),
        w2=pl.BlockSpec((128, CP), lambda i: (0, 0)),
        b2=pl.BlockSpec((1, CP), lambda i: (0, 0)),
        pt=pl.BlockSpec((CP, CP), lambda i: (0, 0)),
    )
    mrow = pl.BlockSpec((MBLK, CP), lambda i: (i, 0))
    mlp = pl.pallas_call(
        _mlp_body,
        grid=(N // MBLK,),
        in_specs=[pl.BlockSpec((MBLK, D), lambda i: (i, 0)),
                  small["w1"], small["b1"], small["w2"], small["b2"],
                  small["pt"]],
        out_specs=[mrow, mrow],
        out_shape=[jax.ShapeDtypeStruct((N, CP), jnp.float32),
                   jax.ShapeDtypeStruct((N, CP), jnp.float32)],
    )
    combine = pl.pallas_call(
        _combine_body,
        out_shape=[jax.ShapeDtypeStruct((NPACK, 128), jnp.float32),
                   jax.ShapeDtypeStruct((NPACK, 128), jnp.float32)],
    )
    return mlp, combine


_MLP, _COMBINE = _build_calls()


@functools.cache
def _edge_call():
    mesh = plsc.VectorSubcoreMesh(core_axis_name="c", subcore_axis_name="s")
    return pl.kernel(
        _edge_body,
        out_type=jax.ShapeDtypeStruct((2, NP, CP), jnp.float32),
        mesh=mesh,
        compiler_params=pltpu.CompilerParams(use_tc_tiling_on_sc=False),
        scratch_types=(
            [pltpu.VMEM((RB, LANE), jnp.int32)] * 8
            + [pltpu.VMEM((RB * LANE, CP), jnp.float32)] * 2
            + [pltpu.VMEM_SHARED((NP, CP), jnp.float32)]
            + [pltpu.SemaphoreType.DMA] * 8
        ),
    )


def kernel(x, edge_index, W1, b1, W2, b2, pairwise_weights):
    src = edge_index[0]
    dst = edge_index[1]
    e = src.shape[0]
    src_r = jnp.concatenate(
        [src, jnp.zeros((E_PAD - e,), jnp.int32)]).reshape(RTOT, LANE)
    dst_r = jnp.concatenate(
        [dst, jnp.full((E_PAD - e,), N, jnp.int32)]).reshape(RTOT, LANE)

    b1r = b1.reshape(1, 128)
    w2p = jnp.pad(W2, ((0, 0), (0, CP - C)))
    b2p = jnp.concatenate([b2, jnp.full((CP - C,), -1e30, jnp.float32)]
                          ).reshape(1, CP)
    pt8 = jnp.zeros((CP, CP), jnp.float32).at[:C, :C].set(
        pairwise_weights.T.astype(jnp.bfloat16).astype(jnp.float32))
    zeros_np = jnp.zeros((NP, CP), jnp.float32)

    l = jnp.arange(128)
    same = (l[:, None] // CP) == (l[None, :] // CP)
    smat = same.astype(jnp.float32)
    bd = jnp.where(same, pt8[l[:, None] % CP, l[None, :] % CP], 0.0)

    logu, logpb = _MLP(x, W1, b1r, w2p, b2p, pt8)
    logu_p = logu.reshape(NPACK, 128)
    edge = _edge_call()
    bel_p = None
    for _ in range(BP_ITERS):
        parts = edge(src_r, dst_r, logpb, zeros_np)
        parts_p = parts.reshape(2, NP // 16, 128)
        bel_p, logpb_p = _COMBINE(logu_p, parts_p, smat, bd)
        logpb = logpb_p.reshape(N, CP)
    return bel_p.reshape(N, CP)[:, :C]


# speedup vs baseline: 37.7717x; 1.0699x over previous
import functools

import jax
import jax.numpy as jnp
from jax import lax
from jax.experimental import pallas as pl
from jax.experimental.pallas import tpu as pltpu
from jax.experimental.pallas import tpu_sc as plsc

N = 100000
D = 256
C = 6
CP = 8
BP_ITERS = 5

NUM_WORKERS = 32
LANE = 128
RB = 14
NSTEP = 28
R_PER_W = NSTEP * RB
RTOT = NUM_WORKERS * R_PER_W
E_PAD = RTOT * LANE
NP = 100352
ROWS_PER_SUB = NP // 16
MBLK = 5000
NPACK = N // 16


def _mlp_body(x_ref, w1_ref, b1_ref, w2_ref, b2_ref, pt_ref, logu_ref, logpb_ref):
    h = jnp.maximum(
        jnp.dot(x_ref[...], w1_ref[...], preferred_element_type=jnp.float32)
        + b1_ref[...], 0.0)
    u = (jnp.dot(h, w2_ref[...], preferred_element_type=jnp.float32)
         + b2_ref[...])
    m = jnp.max(u, axis=1, keepdims=True)
    eu = jnp.exp(u - m)
    s = jnp.sum(eu, axis=1, keepdims=True)
    logu_ref[...] = u - m - jnp.log(s)
    bel = eu / s
    belb = bel.astype(jnp.bfloat16).astype(jnp.float32)
    mm = jnp.zeros_like(bel)
    for c in range(C):
        mm = mm + belb[:, c:c + 1] * pt_ref[c:c + 1, :]
    lane = lax.broadcasted_iota(jnp.int32, mm.shape, 1)
    logpb_ref[...] = jnp.where(lane < C, jnp.log(mm), 0.0)


def _combine_body(logu_ref, parts_ref, smat_ref, bd_ref, bel_ref, logpb_ref):
    lb = logu_ref[...] + parts_ref[0, :NPACK] + parts_ref[1, :NPACK]
    y = jnp.maximum(lb, pltpu.roll(lb, 127, 1))
    y = jnp.maximum(y, pltpu.roll(y, 126, 1))
    y = jnp.maximum(y, pltpu.roll(y, 124, 1))
    lane = lax.broadcasted_iota(jnp.int32, lb.shape, 1)
    leaders = jnp.where(lane % 8 == 0, y, 0.0)
    m = jnp.dot(leaders, smat_ref[...], preferred_element_type=jnp.float32)
    e = jnp.exp(lb - m)
    s = jnp.dot(e, smat_ref[...], preferred_element_type=jnp.float32)
    bel = e / (s + 1e-10)
    bel_ref[...] = bel
    belb = bel.astype(jnp.bfloat16).astype(jnp.float32)
    mm = jnp.dot(belb, bd_ref[...], preferred_element_type=jnp.float32)
    logpb_ref[...] = jnp.where(lane % 8 < C, jnp.log(mm), 0.0)


def _edge_body(src_hbm, dst_hbm, logpb_hbm, zeros_hbm, out_hbm,
               src_q0, src_q1, src_q2, src_q3,
               dst_q0, dst_q1, dst_q2, dst_q3,
               rows_r0, rows_r1, rows_r2, rows_r3, acc_sh,
               isem0, isem1, isem2, isem3,
               gsem0, gsem1, gsem2, gsem3,
               ssem0, ssem1, ssem2, ssem3):
    cid = lax.axis_index("c")
    sid = lax.axis_index("s")
    w = sid * 2 + cid
    src_q = (src_q0, src_q1, src_q2, src_q3)
    dst_q = (dst_q0, dst_q1, dst_q2, dst_q3)
    rows = (rows_r0, rows_r1, rows_r2, rows_r3)
    isem = (isem0, isem1, isem2, isem3)
    gsem = (gsem0, gsem1, gsem2, gsem3)
    ssem = (ssem0, ssem1, ssem2, ssem3)

    pltpu.sync_copy(zeros_hbm.at[pl.ds(sid * ROWS_PER_SUB, ROWS_PER_SUB)],
                    acc_sh.at[pl.ds(sid * ROWS_PER_SUB, ROWS_PER_SUB)])
    plsc.subcore_barrier()

    base_row = w * R_PER_W

    def fire_idx(b, q):
        r0 = base_row + b * RB
        pltpu.async_copy(src_hbm.at[pl.ds(r0, RB)], src_q[q], isem[q])
        pltpu.async_copy(dst_hbm.at[pl.ds(r0, RB)], dst_q[q], isem[q])

    def wait_idx(q):
        pltpu.make_async_copy(src_hbm.at[pl.ds(base_row, RB)],
                              src_q[q], isem[q]).wait()
        pltpu.make_async_copy(dst_hbm.at[pl.ds(base_row, RB)],
                              dst_q[q], isem[q]).wait()

    def fire_gathers(r, q):
        for j in range(RB):
            pltpu.async_copy(logpb_hbm.at[src_q[q].at[j]],
                             rows[r].at[pl.ds(j * LANE, LANE)], gsem[r])

    def drain_gathers(r, q):
        for j in range(RB):
            pltpu.make_async_copy(logpb_hbm.at[src_q[q].at[j]],
                                  rows[r].at[pl.ds(j * LANE, LANE)],
                                  gsem[r]).wait()

    def fire_scatters(r, q):
        for j in range(RB):
            pltpu.async_copy(rows[r].at[pl.ds(j * LANE, LANE)],
                             acc_sh.at[dst_q[q].at[j]], ssem[r], add=True)

    def drain_scatters(r, q):
        for j in range(RB):
            pltpu.make_async_copy(rows[r].at[pl.ds(j * LANE, LANE)],
                                  acc_sh.at[dst_q[q].at[j]], ssem[r]).wait()

    fire_idx(0, 0)
    fire_idx(1, 1)
    wait_idx(0)
    fire_gathers(0, 0)
    wait_idx(1)
    fire_idx(2, 2)
    fire_gathers(1, 1)

    @pl.loop(0, NSTEP // 4)
    def _step(i):
        a = 4 * i
        for k in range(4):
            b = a + k
            q = k % 4
            drain_gathers(q, q)

            @pl.when(b > 0)
            def _():
                drain_scatters((q - 1) % 4, (q - 1) % 4)
            fire_scatters(q, q)

            @pl.when(b + 2 < NSTEP)
            def _():
                wait_idx((q + 2) % 4)

            @pl.when(b + 3 < NSTEP)
            def _():
                fire_idx(b + 3, (q + 3) % 4)

            @pl.when(b + 2 < NSTEP)
            def _():
                fire_gathers((q + 2) % 4, (q + 2) % 4)

    drain_scatters((NSTEP - 1) % 4, (NSTEP - 1) % 4)
    plsc.subcore_barrier()
    pltpu.sync_copy(acc_sh.at[pl.ds(sid * ROWS_PER_SUB, ROWS_PER_SUB)],
                    out_hbm.at[cid, pl.ds(sid * ROWS_PER_SUB, ROWS_PER_SUB)])


def _build_calls():
    small = dict(
        w1=pl.BlockSpec((D, 128), lambda i: (0, 0)),
        b1=pl.BlockSpec((1, 128), lambda i: (0, 0)),
        w2=pl.BlockSpec((128, CP), lambda i: (0, 0)),
        b2=pl.BlockSpec((1, CP), lambda i: (0, 0)),
        pt=pl.BlockSpec((CP, CP), lambda i: (0, 0)),
    )
    mrow = pl.BlockSpec((MBLK, CP), lambda i: (i, 0))
    mlp = pl.pallas_call(
        _mlp_body,
        grid=(N // MBLK,),
        in_specs=[pl.BlockSpec((MBLK, D), lambda i: (i, 0)),
                  small["w1"], small["b1"], small["w2"], small["b2"],
                  small["pt"]],
        out_specs=[mrow, mrow],
        out_shape=[jax.ShapeDtypeStruct((N, CP), jnp.float32),
                   jax.ShapeDtypeStruct((N, CP), jnp.float32)],
    )
    combine = pl.pallas_call(
        _combine_body,
        out_shape=[jax.ShapeDtypeStruct((NPACK, 128), jnp.float32),
                   jax.ShapeDtypeStruct((NPACK, 128), jnp.float32)],
    )
    return mlp, combine


_MLP, _COMBINE = _build_calls()


@functools.cache
def _edge_call():
    mesh = plsc.VectorSubcoreMesh(core_axis_name="c", subcore_axis_name="s")
    return pl.kernel(
        _edge_body,
        out_type=jax.ShapeDtypeStruct((2, NP, CP), jnp.float32),
        mesh=mesh,
        compiler_params=pltpu.CompilerParams(use_tc_tiling_on_sc=False),
        scratch_types=(
            [pltpu.VMEM((RB, LANE), jnp.int32)] * 8
            + [pltpu.VMEM((RB * LANE, CP), jnp.float32)] * 4
            + [pltpu.VMEM_SHARED((NP, CP), jnp.float32)]
            + [pltpu.SemaphoreType.DMA] * 12
        ),
    )


def kernel(x, edge_index, W1, b1, W2, b2, pairwise_weights):
    src = edge_index[0]
    dst = edge_index[1]
    e = src.shape[0]
    src_r = jnp.concatenate(
        [src, jnp.zeros((E_PAD - e,), jnp.int32)]).reshape(RTOT, LANE)
    dst_r = jnp.concatenate(
        [dst, jnp.full((E_PAD - e,), N, jnp.int32)]).reshape(RTOT, LANE)

    b1r = b1.reshape(1, 128)
    w2p = jnp.pad(W2, ((0, 0), (0, CP - C)))
    b2p = jnp.concatenate([b2, jnp.full((CP - C,), -1e30, jnp.float32)]
                          ).reshape(1, CP)
    pt8 = jnp.zeros((CP, CP), jnp.float32).at[:C, :C].set(
        pairwise_weights.T.astype(jnp.bfloat16).astype(jnp.float32))
    zeros_np = jnp.zeros((NP, CP), jnp.float32)

    l = jnp.arange(128)
    same = (l[:, None] // CP) == (l[None, :] // CP)
    smat = same.astype(jnp.float32)
    bd = jnp.where(same, pt8[l[:, None] % CP, l[None, :] % CP], 0.0)

    logu, logpb = _MLP(x, W1, b1r, w2p, b2p, pt8)
    logu_p = logu.reshape(NPACK, 128)
    edge = _edge_call()
    bel_p = None
    for _ in range(BP_ITERS):
        parts = edge(src_r, dst_r, logpb, zeros_np)
        parts_p = parts.reshape(2, NP // 16, 128)
        bel_p, logpb_p = _COMBINE(logu_p, parts_p, smat, bd)
        logpb = logpb_p.reshape(N, CP)
    return bel_p.reshape(N, CP)[:, :C]
